# baseline (speedup 1.0000x reference)
import functools

import jax
import jax.numpy as jnp
from jax import lax
from jax.experimental import pallas as pl
from jax.experimental.pallas import tpu as pltpu
from jax.experimental.pallas import tpu_sc as plsc

NC = 2
NS = 16
NW = NC * NS
EB = 80
LANE = 16

F32 = jnp.float32


def _sc_deg_body(dst_hbm, out_hbm, dst_v, ones_v, fill_v, deg_sh,
                 nchunks, n_pad):
    c = lax.axis_index("c")
    s = lax.axis_index("s")
    wid = s * NC + c
    rpt = n_pad // NS
    one = jnp.full((LANE,), 1.0, dtype=F32)
    for k in range(rpt // LANE):
        fill_v[pl.ds(k * LANE, LANE)] = one
    for k in range(EB // LANE):
        ones_v[pl.ds(k * LANE, LANE)] = one
    pltpu.sync_copy(fill_v, deg_sh.at[pl.ds(s * rpt, rpt)])
    pltpu.sync_copy(dst_hbm.at[wid], dst_v)
    plsc.subcore_barrier()

    def body(j, carry):
        pltpu.sync_copy(ones_v, deg_sh.at[dst_v.at[j]], add=True)
        return carry

    lax.fori_loop(0, nchunks, body, 0)
    plsc.subcore_barrier()
    pltpu.sync_copy(deg_sh.at[pl.ds(s * rpt, rpt)],
                    out_hbm.at[c, pl.ds(s * rpt, rpt)])


def _sc_scatter_body(hp_hbm, src_hbm, dst_hbm, out_hbm,
                     src_v, dst_v, rows_v, acc_sh, sem0, sem1, ssem0, ssem1,
                     nchunks, n_pad):
    c = lax.axis_index("c")
    s = lax.axis_index("s")
    wid = s * NC + c
    rpt = n_pad // NS
    r0 = s * rpt
    pltpu.sync_copy(hp_hbm.at[pl.ds(r0, rpt)], acc_sh.at[pl.ds(r0, rpt)])
    pltpu.sync_copy(src_hbm.at[wid], src_v)
    pltpu.sync_copy(dst_hbm.at[wid], dst_v)
    plsc.subcore_barrier()

    def sgather(j, buf, sem):
        return pltpu.make_async_copy(
            hp_hbm.at[src_v.at[pl.ds(j * EB, EB)]], rows_v.at[buf], sem)

    def sscatter(j, buf, sem):
        return pltpu.make_async_copy(rows_v.at[buf],
                                     acc_sh.at[dst_v.at[j]], sem)

    sgather(0, 0, sem0).start()
    sgather(1, 1, sem1).start()

    def body(jj, carry):
        j0 = 2 * jj
        sgather(j0, 0, sem0).wait()
        sscatter(j0, 0, ssem0).start(add=True)
        sgather(j0 + 1, 1, sem1).wait()
        sscatter(j0 + 1, 1, ssem1).start(add=True)
        sscatter(j0, 0, ssem0).wait()
        sgather(j0 + 2, 0, sem0).start()
        sscatter(j0 + 1, 1, ssem1).wait()
        sgather(j0 + 3, 1, sem1).start()
        return carry

    npairs = (nchunks - 2) // 2
    lax.fori_loop(0, npairs, body, 0)
    jt = 2 * npairs
    sgather(jt, 0, sem0).wait()
    pltpu.sync_copy(rows_v.at[0], acc_sh.at[dst_v.at[jt]], add=True)
    if nchunks % 2 == 1:
        sgather(jt + 2, 0, sem0).start()
    sgather(jt + 1, 1, sem1).wait()
    pltpu.sync_copy(rows_v.at[1], acc_sh.at[dst_v.at[jt + 1]], add=True)
    if nchunks % 2 == 1:
        sgather(jt + 2, 0, sem0).wait()
        pltpu.sync_copy(rows_v.at[0], acc_sh.at[dst_v.at[jt + 2]], add=True)

    plsc.subcore_barrier()
    pltpu.sync_copy(acc_sh.at[pl.ds(r0, rpt)],
                    out_hbm.at[c, pl.ds(r0, rpt)])


def _make_sc_deg(nchunks, n_pad):
    mesh = plsc.VectorSubcoreMesh(core_axis_name="c", subcore_axis_name="s",
                                  num_cores=NC, num_subcores=NS)
    return pl.kernel(
        functools.partial(_sc_deg_body, nchunks=nchunks, n_pad=n_pad),
        out_type=jax.ShapeDtypeStruct((NC, n_pad), F32),
        mesh=mesh,
        scratch_types=[
            pltpu.VMEM((nchunks, EB), jnp.int32),
            pltpu.VMEM((EB,), F32),
            pltpu.VMEM((n_pad // NS,), F32),
            pltpu.VMEM_SHARED((n_pad,), F32),
        ],
    )


def _make_sc_scatter(nchunks, n_pad, d):
    mesh = plsc.VectorSubcoreMesh(core_axis_name="c", subcore_axis_name="s",
                                  num_cores=NC, num_subcores=NS)
    return pl.kernel(
        functools.partial(_sc_scatter_body, nchunks=nchunks, n_pad=n_pad),
        out_type=jax.ShapeDtypeStruct((NC, n_pad, d), F32),
        mesh=mesh,
        scratch_types=[
            pltpu.VMEM((nchunks * EB,), jnp.int32),
            pltpu.VMEM((nchunks, EB), jnp.int32),
            pltpu.VMEM((2, EB, d), F32),
            pltpu.VMEM_SHARED((n_pad, d), F32),
            pltpu.SemaphoreType.DMA,
            pltpu.SemaphoreType.DMA,
            pltpu.SemaphoreType.DMA,
            pltpu.SemaphoreType.DMA,
        ],
    )


def _tc_first_body(x_ref, w_ref, deg_ref, dinv_ref, hp_ref):
    deg = deg_ref[0] + deg_ref[1] - 1.0
    dv = lax.rsqrt(deg)
    dinv_ref[...] = dv
    h = jnp.dot(x_ref[...], w_ref[...], preferred_element_type=F32)
    hp_ref[...] = h * dv


def _tc_mid_body(p_ref, hp_ref, dinv_ref, b_ref, w_ref, xk_ref, hn_ref):
    dv = dinv_ref[...]
    xk = dv * (p_ref[0] + p_ref[1] - hp_ref[...]) + b_ref[...]
    xk_ref[...] = xk
    hn_ref[...] = jnp.dot(xk, w_ref[...], preferred_element_type=F32) * dv


def _tc_final_body(p_ref, hp_ref, dinv_ref, b_ref,
                   x_ref, x1_ref, x2_ref, wl_ref, bl_ref,
                   hcat_ref, pred_ref):
    dv = dinv_ref[...]
    x3 = dv * (p_ref[0] + p_ref[1] - hp_ref[...]) + b_ref[...]
    hcat = jnp.concatenate([x_ref[...], x1_ref[...], x2_ref[...], x3], axis=-1)
    hcat_ref[...] = hcat
    pred_ref[...] = (jnp.dot(hcat, wl_ref[...], preferred_element_type=F32)
                     + bl_ref[...])


def _row_spec(rb, cols):
    return pl.BlockSpec((rb, cols), lambda i: (i, 0))


def _p_spec(rb, cols):
    return pl.BlockSpec((2, rb, cols), lambda i: (0, i, 0))


def _bcast_spec(rows, cols):
    return pl.BlockSpec((rows, cols), lambda i: (0, 0))


def _tc_first(x, w, deg2, n, n_pad, d, rb):
    grid = (n // rb,)
    return pl.pallas_call(
        _tc_first_body,
        grid=grid,
        in_specs=[_row_spec(rb, d), _bcast_spec(d, d),
                  pl.BlockSpec((2, rb, 1), lambda i: (0, i, 0))],
        out_specs=[_row_spec(rb, 1), _row_spec(rb, d)],
        out_shape=[jax.ShapeDtypeStruct((n_pad, 1), F32),
                   jax.ShapeDtypeStruct((n_pad, d), F32)],
    )(x, w, deg2)


def _tc_mid(p, hp, dinv, b, w, n, n_pad, d, rb):
    grid = (n // rb,)
    return pl.pallas_call(
        _tc_mid_body,
        grid=grid,
        in_specs=[_p_spec(rb, d), _row_spec(rb, d),
                  _row_spec(rb, 1), _bcast_spec(1, d), _bcast_spec(d, d)],
        out_specs=[_row_spec(rb, d), _row_spec(rb, d)],
        out_shape=[jax.ShapeDtypeStruct((n_pad, d), F32),
                   jax.ShapeDtypeStruct((n_pad, d), F32)],
    )(p, hp, dinv, b, w)


def _tc_final(p, hp, dinv, b, x, x1, x2, wl, bl, n, d, c_out, rb):
    grid = (n // rb,)
    dcat = 4 * d
    return pl.pallas_call(
        _tc_final_body,
        grid=grid,
        in_specs=[_p_spec(rb, d), _row_spec(rb, d),
                  _row_spec(rb, 1), _bcast_spec(1, d),
                  _row_spec(rb, d), _row_spec(rb, d), _row_spec(rb, d),
                  _bcast_spec(dcat, c_out), _bcast_spec(1, c_out)],
        out_specs=[_row_spec(rb, dcat), _row_spec(rb, c_out)],
        out_shape=[jax.ShapeDtypeStruct((n, dcat), F32),
                   jax.ShapeDtypeStruct((n, c_out), F32)],
    )(p, hp, dinv, b, x, x1, x2, wl, bl)


def kernel(x, edge_index, W1, b1, W2, b2, W3, b3, Wlin, blin):
    n, d = x.shape
    c_out = Wlin.shape[1]
    e = edge_index.shape[1]

    n_pad = -(-n // (NS * 8 * NC)) * (NS * 8 * NC)
    nchunks = max(2, -(-e // (NW * EB)))
    e_pad = NW * EB * nchunks
    rb = 1000
    assert n % rb == 0

    src = edge_index[0]
    dst = edge_index[1]
    if e_pad > e:
        npad_e = e_pad - e
        pad_rows = n + (jnp.arange(npad_e, dtype=jnp.int32) % (n_pad - n))
        src = jnp.concatenate([src, pad_rows])
        dst = jnp.concatenate([dst, pad_rows])
    src = src.reshape(NW, nchunks * EB)
    dst = dst.reshape(NW, nchunks, EB)

    sc_deg = _make_sc_deg(nchunks, n_pad)
    sc_scatter = _make_sc_scatter(nchunks, n_pad, d)

    deg2 = sc_deg(dst)[:, :, None]

    b1r = b1[None, :]
    b2r = b2[None, :]
    b3r = b3[None, :]
    blr = blin[None, :]

    dinv, h1p = _tc_first(x, W1, deg2, n, n_pad, d, rb)
    p = sc_scatter(h1p, src, dst)
    x1, h2p = _tc_mid(p, h1p, dinv, b1r, W2, n, n_pad, d, rb)
    p = sc_scatter(h2p, src, dst)
    x2, h3p = _tc_mid(p, h2p, dinv, b2r, W3, n, n_pad, d, rb)
    p = sc_scatter(h3p, src, dst)
    hcat, pred = _tc_final(p, h3p, dinv, b3r, x, x1, x2,
                           Wlin, blr, n, d, c_out, rb)
    return (pred, hcat)


# speedup vs baseline: 1.2209x; 1.2209x over previous
import functools

import jax
import jax.numpy as jnp
from jax import lax
from jax.experimental import pallas as pl
from jax.experimental.pallas import tpu as pltpu
from jax.experimental.pallas import tpu_sc as plsc

NC = 2
NS = 16
NW = NC * NS
EB = 80
LANE = 16

F32 = jnp.float32


def _sc_deg_body(dst_hbm, out_hbm, dst_v, ones_v, fill_v, deg_sh,
                 nchunks, n_pad):
    c = lax.axis_index("c")
    s = lax.axis_index("s")
    wid = s * NC + c
    rpt = n_pad // NS
    one = jnp.full((LANE,), 1.0, dtype=F32)
    for k in range(rpt // LANE):
        fill_v[pl.ds(k * LANE, LANE)] = one
    for k in range(EB // LANE):
        ones_v[pl.ds(k * LANE, LANE)] = one
    pltpu.sync_copy(fill_v, deg_sh.at[pl.ds(s * rpt, rpt)])
    pltpu.sync_copy(dst_hbm.at[wid], dst_v)
    plsc.subcore_barrier()

    def body(j, carry):
        pltpu.sync_copy(ones_v, deg_sh.at[dst_v.at[j]], add=True)
        return carry

    lax.fori_loop(0, nchunks, body, 0)
    plsc.subcore_barrier()
    pltpu.sync_copy(deg_sh.at[pl.ds(s * rpt, rpt)],
                    out_hbm.at[c, pl.ds(s * rpt, rpt)])


def _sc_scatter_body(hp_hbm, src_hbm, dst_hbm, out_hbm,
                     src_v, dst_v, rows_v, acc_sh, sem0, sem1,
                     nchunks, n_pad):
    c = lax.axis_index("c")
    s = lax.axis_index("s")
    wid = s * NC + c
    rpt = n_pad // NS
    r0 = s * rpt
    pltpu.sync_copy(hp_hbm.at[pl.ds(r0, rpt)], acc_sh.at[pl.ds(r0, rpt)])
    pltpu.sync_copy(src_hbm.at[wid], src_v)
    pltpu.sync_copy(dst_hbm.at[wid], dst_v)
    plsc.subcore_barrier()

    def sgather(j, buf, sem):
        return pltpu.make_async_copy(
            hp_hbm.at[src_v.at[pl.ds(j * EB, EB)]], rows_v.at[buf], sem)

    sgather(0, 0, sem0).start()
    sgather(1, 1, sem1).start()

    def body(jj, carry):
        j0 = 2 * jj
        sgather(j0, 0, sem0).wait()
        pltpu.sync_copy(rows_v.at[0], acc_sh.at[dst_v.at[j0]], add=True)
        sgather(j0 + 2, 0, sem0).start()
        sgather(j0 + 1, 1, sem1).wait()
        pltpu.sync_copy(rows_v.at[1], acc_sh.at[dst_v.at[j0 + 1]], add=True)
        sgather(j0 + 3, 1, sem1).start()
        return carry

    npairs = (nchunks - 2) // 2
    lax.fori_loop(0, npairs, body, 0)
    jt = 2 * npairs
    sgather(jt, 0, sem0).wait()
    pltpu.sync_copy(rows_v.at[0], acc_sh.at[dst_v.at[jt]], add=True)
    if nchunks % 2 == 1:
        sgather(jt + 2, 0, sem0).start()
    sgather(jt + 1, 1, sem1).wait()
    pltpu.sync_copy(rows_v.at[1], acc_sh.at[dst_v.at[jt + 1]], add=True)
    if nchunks % 2 == 1:
        sgather(jt + 2, 0, sem0).wait()
        pltpu.sync_copy(rows_v.at[0], acc_sh.at[dst_v.at[jt + 2]], add=True)

    plsc.subcore_barrier()
    pltpu.sync_copy(acc_sh.at[pl.ds(r0, rpt)],
                    out_hbm.at[c, pl.ds(r0, rpt)])


def _make_sc_deg(nchunks, n_pad):
    mesh = plsc.VectorSubcoreMesh(core_axis_name="c", subcore_axis_name="s",
                                  num_cores=NC, num_subcores=NS)
    return pl.kernel(
        functools.partial(_sc_deg_body, nchunks=nchunks, n_pad=n_pad),
        out_type=jax.ShapeDtypeStruct((NC, n_pad), F32),
        mesh=mesh,
        scratch_types=[
            pltpu.VMEM((nchunks, EB), jnp.int32),
            pltpu.VMEM((EB,), F32),
            pltpu.VMEM((n_pad // NS,), F32),
            pltpu.VMEM_SHARED((n_pad,), F32),
        ],
    )


def _make_sc_scatter(nchunks, n_pad, d):
    mesh = plsc.VectorSubcoreMesh(core_axis_name="c", subcore_axis_name="s",
                                  num_cores=NC, num_subcores=NS)
    return pl.kernel(
        functools.partial(_sc_scatter_body, nchunks=nchunks, n_pad=n_pad),
        out_type=jax.ShapeDtypeStruct((NC, n_pad, d), F32),
        mesh=mesh,
        scratch_types=[
            pltpu.VMEM((nchunks * EB,), jnp.int32),
            pltpu.VMEM((nchunks, EB), jnp.int32),
            pltpu.VMEM((2, EB, d), F32),
            pltpu.VMEM_SHARED((n_pad, d), F32),
            pltpu.SemaphoreType.DMA,
            pltpu.SemaphoreType.DMA,
        ],
    )


def _tc_matmul_body(x_ref, w_ref, h_ref):
    h_ref[...] = jnp.dot(x_ref[...], w_ref[...], preferred_element_type=F32)


def _tc_scale_body(h_ref, deg_ref, dinv_ref, hp_ref):
    deg = deg_ref[0] + deg_ref[1] - 1.0
    dv = lax.rsqrt(deg)
    dinv_ref[...] = dv
    hp_ref[...] = h_ref[...] * dv


def _tc_mid_body(p_ref, hp_ref, dinv_ref, b_ref, w_ref, xk_ref, hn_ref):
    dv = dinv_ref[...]
    xk = dv * (p_ref[0] + p_ref[1] - hp_ref[...]) + b_ref[...]
    xk_ref[...] = xk
    hn_ref[...] = jnp.dot(xk, w_ref[...], preferred_element_type=F32) * dv


def _tc_final_body(p_ref, hp_ref, dinv_ref, b_ref,
                   x_ref, x1_ref, x2_ref, wl_ref, bl_ref,
                   hcat_ref, pred_ref):
    dv = dinv_ref[...]
    x3 = dv * (p_ref[0] + p_ref[1] - hp_ref[...]) + b_ref[...]
    hcat = jnp.concatenate([x_ref[...], x1_ref[...], x2_ref[...], x3], axis=-1)
    hcat_ref[...] = hcat
    pred_ref[...] = (jnp.dot(hcat, wl_ref[...], preferred_element_type=F32)
                     + bl_ref[...])


def _row_spec(rb, cols):
    return pl.BlockSpec((rb, cols), lambda i: (i, 0))


def _p_spec(rb, cols):
    return pl.BlockSpec((2, rb, cols), lambda i: (0, i, 0))


def _bcast_spec(rows, cols):
    return pl.BlockSpec((rows, cols), lambda i: (0, 0))


def _tc_matmul(x, w, n, d, rb):
    grid = (n // rb,)
    return pl.pallas_call(
        _tc_matmul_body,
        grid=grid,
        in_specs=[_row_spec(rb, d), _bcast_spec(d, d)],
        out_specs=_row_spec(rb, d),
        out_shape=jax.ShapeDtypeStruct((n, d), F32),
    )(x, w)


def _tc_scale(h, deg2, n, n_pad, d, rb):
    grid = (n // rb,)
    return pl.pallas_call(
        _tc_scale_body,
        grid=grid,
        in_specs=[_row_spec(rb, d),
                  pl.BlockSpec((2, rb, 1), lambda i: (0, i, 0))],
        out_specs=[_row_spec(rb, 1), _row_spec(rb, d)],
        out_shape=[jax.ShapeDtypeStruct((n_pad, 1), F32),
                   jax.ShapeDtypeStruct((n_pad, d), F32)],
    )(h, deg2)


def _tc_mid(p, hp, dinv, b, w, n, n_pad, d, rb):
    grid = (n // rb,)
    return pl.pallas_call(
        _tc_mid_body,
        grid=grid,
        in_specs=[_p_spec(rb, d), _row_spec(rb, d),
                  _row_spec(rb, 1), _bcast_spec(1, d), _bcast_spec(d, d)],
        out_specs=[_row_spec(rb, d), _row_spec(rb, d)],
        out_shape=[jax.ShapeDtypeStruct((n_pad, d), F32),
                   jax.ShapeDtypeStruct((n_pad, d), F32)],
    )(p, hp, dinv, b, w)


def _tc_final(p, hp, dinv, b, x, x1, x2, wl, bl, n, d, c_out, rb):
    grid = (n // rb,)
    dcat = 4 * d
    return pl.pallas_call(
        _tc_final_body,
        grid=grid,
        in_specs=[_p_spec(rb, d), _row_spec(rb, d),
                  _row_spec(rb, 1), _bcast_spec(1, d),
                  _row_spec(rb, d), _row_spec(rb, d), _row_spec(rb, d),
                  _bcast_spec(dcat, c_out), _bcast_spec(1, c_out)],
        out_specs=[_row_spec(rb, dcat), _row_spec(rb, c_out)],
        out_shape=[jax.ShapeDtypeStruct((n, dcat), F32),
                   jax.ShapeDtypeStruct((n, c_out), F32)],
    )(p, hp, dinv, b, x, x1, x2, wl, bl)


def kernel(x, edge_index, W1, b1, W2, b2, W3, b3, Wlin, blin):
    n, d = x.shape
    c_out = Wlin.shape[1]
    e = edge_index.shape[1]

    n_pad = -(-n // (NS * 8 * NC)) * (NS * 8 * NC)
    nchunks = max(2, -(-e // (NW * EB)))
    e_pad = NW * EB * nchunks
    rb = 1000
    assert n % rb == 0

    src = edge_index[0]
    dst = edge_index[1]
    if e_pad > e:
        npad_e = e_pad - e
        pad_rows = n + (jnp.arange(npad_e, dtype=jnp.int32) % (n_pad - n))
        src = jnp.concatenate([src, pad_rows])
        dst = jnp.concatenate([dst, pad_rows])
    src = src.reshape(NW, nchunks * EB)
    dst = dst.reshape(NW, nchunks, EB)

    sc_deg = _make_sc_deg(nchunks, n_pad)
    sc_scatter = _make_sc_scatter(nchunks, n_pad, d)

    deg2 = sc_deg(dst)[:, :, None]

    b1r = b1[None, :]
    b2r = b2[None, :]
    b3r = b3[None, :]
    blr = blin[None, :]

    h1 = _tc_matmul(x, W1, n, d, rb)
    dinv, h1p = _tc_scale(h1, deg2, n, n_pad, d, rb)
    p = sc_scatter(h1p, src, dst)
    x1, h2p = _tc_mid(p, h1p, dinv, b1r, W2, n, n_pad, d, rb)
    p = sc_scatter(h2p, src, dst)
    x2, h3p = _tc_mid(p, h2p, dinv, b2r, W3, n, n_pad, d, rb)
    p = sc_scatter(h3p, src, dst)
    hcat, pred = _tc_final(p, h3p, dinv, b3r, x, x1, x2,
                           Wlin, blr, n, d, c_out, rb)
    return (pred, hcat)


# speedup vs baseline: 1.2645x; 1.0357x over previous
import functools

import jax
import jax.numpy as jnp
from jax import lax
from jax.experimental import pallas as pl
from jax.experimental.pallas import tpu as pltpu
from jax.experimental.pallas import tpu_sc as plsc

NC = 2
NS = 16
NW = NC * NS
EB = 96
LANE = 16

F32 = jnp.float32


def _sc_deg_body(dst_hbm, out_hbm, dst_v, ones_v, fill_v, deg_sh,
                 nchunks, n_pad):
    c = lax.axis_index("c")
    s = lax.axis_index("s")
    wid = s * NC + c
    rpt = n_pad // NS
    one = jnp.full((LANE,), 1.0, dtype=F32)
    for k in range(rpt // LANE):
        fill_v[pl.ds(k * LANE, LANE)] = one
    for k in range(EB // LANE):
        ones_v[pl.ds(k * LANE, LANE)] = one
    pltpu.sync_copy(fill_v, deg_sh.at[pl.ds(s * rpt, rpt)])
    pltpu.sync_copy(dst_hbm.at[wid], dst_v)
    plsc.subcore_barrier()

    def body(j, carry):
        pltpu.sync_copy(ones_v, deg_sh.at[dst_v.at[j]], add=True)
        return carry

    lax.fori_loop(0, nchunks, body, 0)
    plsc.subcore_barrier()
    pltpu.sync_copy(deg_sh.at[pl.ds(s * rpt, rpt)],
                    out_hbm.at[c, pl.ds(s * rpt, rpt)])


def _sc_scatter_body(hp_hbm, src_hbm, dst_hbm, out_hbm,
                     src_v, dst_v, rows_v, acc_sh, sem0, sem1,
                     nchunks, n_pad):
    c = lax.axis_index("c")
    s = lax.axis_index("s")
    wid = s * NC + c
    rpt = n_pad // NS
    r0 = s * rpt
    pltpu.sync_copy(hp_hbm.at[pl.ds(r0, rpt)], acc_sh.at[pl.ds(r0, rpt)])
    pltpu.sync_copy(src_hbm.at[wid], src_v)
    pltpu.sync_copy(dst_hbm.at[wid], dst_v)
    plsc.subcore_barrier()

    def sgather(j, buf, sem):
        return pltpu.make_async_copy(
            hp_hbm.at[src_v.at[pl.ds(j * EB, EB)]], rows_v.at[buf], sem)

    sgather(0, 0, sem0).start()
    sgather(1, 1, sem1).start()

    def body(jj, carry):
        j0 = 2 * jj
        sgather(j0, 0, sem0).wait()
        pltpu.sync_copy(rows_v.at[0], acc_sh.at[dst_v.at[j0]], add=True)
        sgather(j0 + 2, 0, sem0).start()
        sgather(j0 + 1, 1, sem1).wait()
        pltpu.sync_copy(rows_v.at[1], acc_sh.at[dst_v.at[j0 + 1]], add=True)
        sgather(j0 + 3, 1, sem1).start()
        return carry

    npairs = (nchunks - 2) // 2
    lax.fori_loop(0, npairs, body, 0)
    jt = 2 * npairs
    sgather(jt, 0, sem0).wait()
    pltpu.sync_copy(rows_v.at[0], acc_sh.at[dst_v.at[jt]], add=True)
    if nchunks % 2 == 1:
        sgather(jt + 2, 0, sem0).start()
    sgather(jt + 1, 1, sem1).wait()
    pltpu.sync_copy(rows_v.at[1], acc_sh.at[dst_v.at[jt + 1]], add=True)
    if nchunks % 2 == 1:
        sgather(jt + 2, 0, sem0).wait()
        pltpu.sync_copy(rows_v.at[0], acc_sh.at[dst_v.at[jt + 2]], add=True)

    plsc.subcore_barrier()
    pltpu.sync_copy(acc_sh.at[pl.ds(r0, rpt)],
                    out_hbm.at[c, pl.ds(r0, rpt)])


def _make_sc_deg(nchunks, n_pad):
    mesh = plsc.VectorSubcoreMesh(core_axis_name="c", subcore_axis_name="s",
                                  num_cores=NC, num_subcores=NS)
    return pl.kernel(
        functools.partial(_sc_deg_body, nchunks=nchunks, n_pad=n_pad),
        out_type=jax.ShapeDtypeStruct((NC, n_pad), F32),
        mesh=mesh,
        scratch_types=[
            pltpu.VMEM((nchunks, EB), jnp.int32),
            pltpu.VMEM((EB,), F32),
            pltpu.VMEM((n_pad // NS,), F32),
            pltpu.VMEM_SHARED((n_pad,), F32),
        ],
    )


def _make_sc_scatter(nchunks, n_pad, d):
    mesh = plsc.VectorSubcoreMesh(core_axis_name="c", subcore_axis_name="s",
                                  num_cores=NC, num_subcores=NS)
    return pl.kernel(
        functools.partial(_sc_scatter_body, nchunks=nchunks, n_pad=n_pad),
        out_type=jax.ShapeDtypeStruct((NC, n_pad, d), F32),
        mesh=mesh,
        scratch_types=[
            pltpu.VMEM((nchunks * EB,), jnp.int32),
            pltpu.VMEM((nchunks, EB), jnp.int32),
            pltpu.VMEM((2, EB, d), F32),
            pltpu.VMEM_SHARED((n_pad, d), F32),
            pltpu.SemaphoreType.DMA,
            pltpu.SemaphoreType.DMA,
        ],
    )


def _tc_matmul_body(x_ref, w_ref, h_ref):
    h_ref[...] = jnp.dot(x_ref[...], w_ref[...], preferred_element_type=F32)


def _tc_scale_body(h_ref, deg_ref, dinv_ref, hp_ref):
    deg = deg_ref[0] + deg_ref[1] - 1.0
    dv = lax.rsqrt(deg)
    dinv_ref[...] = dv
    hp_ref[...] = h_ref[...] * dv


def _tc_mid_body(p_ref, hp_ref, dinv_ref, b_ref, w_ref, xk_ref, hn_ref):
    dv = dinv_ref[...]
    xk = dv * (p_ref[0] + p_ref[1] - hp_ref[...]) + b_ref[...]
    xk_ref[...] = xk
    hn_ref[...] = jnp.dot(xk, w_ref[...], preferred_element_type=F32) * dv


def _tc_final_body(p_ref, hp_ref, dinv_ref, b_ref,
                   x_ref, x1_ref, x2_ref, wl_ref, bl_ref,
                   hcat_ref, pred_ref):
    dv = dinv_ref[...]
    x3 = dv * (p_ref[0] + p_ref[1] - hp_ref[...]) + b_ref[...]
    hcat = jnp.concatenate([x_ref[...], x1_ref[...], x2_ref[...], x3], axis=-1)
    hcat_ref[...] = hcat
    pred_ref[...] = (jnp.dot(hcat, wl_ref[...], preferred_element_type=F32)
                     + bl_ref[...])


def _row_spec(rb, cols):
    return pl.BlockSpec((rb, cols), lambda i: (i, 0))


def _p_spec(rb, cols):
    return pl.BlockSpec((2, rb, cols), lambda i: (0, i, 0))


def _bcast_spec(rows, cols):
    return pl.BlockSpec((rows, cols), lambda i: (0, 0))


def _tc_matmul(x, w, n, d, rb):
    grid = (n // rb,)
    return pl.pallas_call(
        _tc_matmul_body,
        grid=grid,
        in_specs=[_row_spec(rb, d), _bcast_spec(d, d)],
        out_specs=_row_spec(rb, d),
        out_shape=jax.ShapeDtypeStruct((n, d), F32),
    )(x, w)


def _tc_scale(h, deg2, n, n_pad, d, rb):
    grid = (n // rb,)
    return pl.pallas_call(
        _tc_scale_body,
        grid=grid,
        in_specs=[_row_spec(rb, d),
                  pl.BlockSpec((2, rb, 1), lambda i: (0, i, 0))],
        out_specs=[_row_spec(rb, 1), _row_spec(rb, d)],
        out_shape=[jax.ShapeDtypeStruct((n_pad, 1), F32),
                   jax.ShapeDtypeStruct((n_pad, d), F32)],
    )(h, deg2)


def _tc_mid(p, hp, dinv, b, w, n, n_pad, d, rb):
    grid = (n // rb,)
    return pl.pallas_call(
        _tc_mid_body,
        grid=grid,
        in_specs=[_p_spec(rb, d), _row_spec(rb, d),
                  _row_spec(rb, 1), _bcast_spec(1, d), _bcast_spec(d, d)],
        out_specs=[_row_spec(rb, d), _row_spec(rb, d)],
        out_shape=[jax.ShapeDtypeStruct((n_pad, d), F32),
                   jax.ShapeDtypeStruct((n_pad, d), F32)],
    )(p, hp, dinv, b, w)


def _tc_final(p, hp, dinv, b, x, x1, x2, wl, bl, n, d, c_out, rb):
    grid = (n // rb,)
    dcat = 4 * d
    return pl.pallas_call(
        _tc_final_body,
        grid=grid,
        in_specs=[_p_spec(rb, d), _row_spec(rb, d),
                  _row_spec(rb, 1), _bcast_spec(1, d),
                  _row_spec(rb, d), _row_spec(rb, d), _row_spec(rb, d),
                  _bcast_spec(dcat, c_out), _bcast_spec(1, c_out)],
        out_specs=[_row_spec(rb, dcat), _row_spec(rb, c_out)],
        out_shape=[jax.ShapeDtypeStruct((n, dcat), F32),
                   jax.ShapeDtypeStruct((n, c_out), F32)],
    )(p, hp, dinv, b, x, x1, x2, wl, bl)


def kernel(x, edge_index, W1, b1, W2, b2, W3, b3, Wlin, blin):
    n, d = x.shape
    c_out = Wlin.shape[1]
    e = edge_index.shape[1]

    n_pad = -(-n // (NS * 8 * NC)) * (NS * 8 * NC)
    nchunks = max(2, -(-e // (NW * EB)))
    e_pad = NW * EB * nchunks
    rb = 1000
    assert n % rb == 0

    src = edge_index[0]
    dst = edge_index[1]
    if e_pad > e:
        npad_e = e_pad - e
        pad_rows = n + (jnp.arange(npad_e, dtype=jnp.int32) % (n_pad - n))
        src = jnp.concatenate([src, pad_rows])
        dst = jnp.concatenate([dst, pad_rows])
    src = src.reshape(NW, nchunks * EB)
    dst = dst.reshape(NW, nchunks, EB)

    sc_deg = _make_sc_deg(nchunks, n_pad)
    sc_scatter = _make_sc_scatter(nchunks, n_pad, d)

    deg2 = sc_deg(dst)[:, :, None]

    b1r = b1[None, :]
    b2r = b2[None, :]
    b3r = b3[None, :]
    blr = blin[None, :]

    h1 = _tc_matmul(x, W1, n, d, rb)
    dinv, h1p = _tc_scale(h1, deg2, n, n_pad, d, rb)
    p = sc_scatter(h1p, src, dst)
    x1, h2p = _tc_mid(p, h1p, dinv, b1r, W2, n, n_pad, d, rb)
    p = sc_scatter(h2p, src, dst)
    x2, h3p = _tc_mid(p, h2p, dinv, b2r, W3, n, n_pad, d, rb)
    p = sc_scatter(h3p, src, dst)
    hcat, pred = _tc_final(p, h3p, dinv, b3r, x, x1, x2,
                           Wlin, blr, n, d, c_out, rb)
    return (pred, hcat)
